# baseline (speedup 1.0000x reference)
import functools

import jax
import jax.numpy as jnp
import numpy as np
from jax.experimental import pallas as pl
from jax.experimental.pallas import tpu as pltpu

D = 768
S = 2048
B = 4
K_KEEP = 614
_SCALE = 1.0 / np.sqrt(np.float32(D))

_RQ = 512
_RA = 256


def _qkv_body(x_ref, w_ref, b_ref, g_ref, be_ref, xn_ref, q_ref, k_ref, v_ref):
    x = x_ref[...]
    mu = jnp.mean(x, axis=-1, keepdims=True)
    var = jnp.mean((x - mu) * (x - mu), axis=-1, keepdims=True)
    xn = (x - mu) * jax.lax.rsqrt(var + 1e-5) * g_ref[...] + be_ref[...]
    xn_ref[...] = xn
    qkv = jax.lax.dot_general(xn.astype(jnp.bfloat16), w_ref[...],
                              (((1,), (1,)), ((), ())),
                              preferred_element_type=jnp.float32) + b_ref[...]
    q_ref[...] = qkv[:, :D].astype(jnp.bfloat16)
    k_ref[...] = qkv[:, D:2 * D].astype(jnp.bfloat16)
    v_ref[...] = qkv[:, 2 * D:].astype(jnp.bfloat16)


def _attn_body(q_ref, k_ref, v_ref, xn_ref, wo_ref, bo_ref, o_ref):
    q = q_ref[0]
    k = k_ref[0]
    s = jax.lax.dot_general(q, k, (((1,), (1,)), ((), ())),
                            preferred_element_type=jnp.float32) * _SCALE
    y = jax.lax.bitcast_convert_type(s, jnp.int32)
    y = jnp.where(y < 0, y ^ jnp.int32(0x7FFFFFFF), y)

    def _unmap(t):
        return jax.lax.bitcast_convert_type(
            jnp.where(t < 0, t ^ jnp.int32(0x7FFFFFFF), t), jnp.float32)

    lo = jnp.min(y, axis=-1, keepdims=True)
    hi = jnp.max(y, axis=-1, keepdims=True) + 1
    cl = jnp.full_like(lo, S)
    ch = jnp.zeros_like(lo)

    def cond(carry):
        _, lo, hi, _, _ = carry
        return jnp.max(((hi - lo) != 1).astype(jnp.int32)) > 0

    def body(carry):
        it, lo, hi, cl, ch = carry
        lo_f = _unmap(lo)
        hi_f = _unmap(hi)
        frac = (cl - K_KEEP).astype(jnp.float32) / (cl - ch).astype(jnp.float32)
        mid_f = lo_f + (hi_f - lo_f) * frac
        mid_i = jax.lax.bitcast_convert_type(mid_f, jnp.int32)
        mid_interp = jnp.where(mid_i < 0, mid_i ^ jnp.int32(0x7FFFFFFF), mid_i)
        mid_bisect = (lo >> 1) + (hi >> 1) + ((lo | hi) & 1)
        mid = jnp.where((it & 1) == 0, mid_interp, mid_bisect)
        mid = jnp.minimum(jnp.maximum(mid, lo + 1), hi - 1)
        cnt = jnp.sum((y >= mid).astype(jnp.int32), axis=-1, keepdims=True)
        ge = cnt >= K_KEEP
        lo = jnp.where(ge, mid, lo)
        cl = jnp.where(ge, cnt, cl)
        hi = jnp.where(ge, hi, mid)
        ch = jnp.where(ge, ch, cnt)
        return it + 1, lo, hi, cl, ch

    _, lo, hi, cl, ch = jax.lax.while_loop(cond, body, (0, lo, hi, cl, ch))

    mask = y >= lo
    m = jnp.max(s, axis=-1, keepdims=True)
    p = jnp.where(mask, jnp.exp(s - m), 0.0)
    z = jnp.sum(p, axis=-1, keepdims=True)
    w = (p / z).astype(jnp.bfloat16)
    attn = jax.lax.dot_general(w, v_ref[0], (((1,), (0,)), ((), ())),
                               preferred_element_type=jnp.float32)
    out = jax.lax.dot_general(attn.astype(jnp.bfloat16), wo_ref[...],
                              (((1,), (1,)), ((), ())),
                              preferred_element_type=jnp.float32) + bo_ref[...]
    o_ref[0] = out + xn_ref[0]


@jax.jit
def kernel(x, Wqkv, bqkv, Wout, bout, gamma, beta):
    xf = x.reshape(B * S, D)
    xn, q, k, v = pl.pallas_call(
        _qkv_body,
        grid=(B * S // _RQ,),
        in_specs=[
            pl.BlockSpec((_RQ, D), lambda i: (i, 0)),
            pl.BlockSpec((3 * D, D), lambda i: (0, 0)),
            pl.BlockSpec((1, 3 * D), lambda i: (0, 0)),
            pl.BlockSpec((1, D), lambda i: (0, 0)),
            pl.BlockSpec((1, D), lambda i: (0, 0)),
        ],
        out_specs=[
            pl.BlockSpec((_RQ, D), lambda i: (i, 0)),
            pl.BlockSpec((_RQ, D), lambda i: (i, 0)),
            pl.BlockSpec((_RQ, D), lambda i: (i, 0)),
            pl.BlockSpec((_RQ, D), lambda i: (i, 0)),
        ],
        out_shape=[jax.ShapeDtypeStruct((B * S, D), jnp.float32)] +
                  [jax.ShapeDtypeStruct((B * S, D), jnp.bfloat16)] * 3,
    )(xf, Wqkv.astype(jnp.bfloat16), bqkv.reshape(1, 3 * D),
      gamma.reshape(1, D), beta.reshape(1, D))

    q = q.reshape(B, S, D)
    k = k.reshape(B, S, D)
    v = v.reshape(B, S, D)
    xn = xn.reshape(B, S, D)

    out = pl.pallas_call(
        _attn_body,
        grid=(B, S // _RA),
        in_specs=[
            pl.BlockSpec((1, _RA, D), lambda b, i: (b, i, 0)),
            pl.BlockSpec((1, S, D), lambda b, i: (b, 0, 0)),
            pl.BlockSpec((1, S, D), lambda b, i: (b, 0, 0)),
            pl.BlockSpec((1, _RA, D), lambda b, i: (b, i, 0)),
            pl.BlockSpec((D, D), lambda b, i: (0, 0)),
            pl.BlockSpec((1, D), lambda b, i: (0, 0)),
        ],
        out_specs=pl.BlockSpec((1, _RA, D), lambda b, i: (b, i, 0)),
        out_shape=jax.ShapeDtypeStruct((B, S, D), jnp.float32),
    )(q, k, v, xn, Wout.astype(jnp.bfloat16), bout.reshape(1, D))

    return out


# speedup vs baseline: 2.0539x; 2.0539x over previous
import functools

import jax
import jax.numpy as jnp
import numpy as np
from jax.experimental import pallas as pl
from jax.experimental.pallas import tpu as pltpu

D = 768
S = 2048
B = 4
K_KEEP = 614
_SCALE = 1.0 / np.sqrt(np.float32(D))

_RQ = 512
_RA = 256


def _qkv_body(x_ref, w_ref, b_ref, g_ref, be_ref, xn_ref, q_ref, k_ref, v_ref):
    x = x_ref[...]
    mu = jnp.mean(x, axis=-1, keepdims=True)
    var = jnp.mean((x - mu) * (x - mu), axis=-1, keepdims=True)
    xn = (x - mu) * jax.lax.rsqrt(var + 1e-5) * g_ref[...] + be_ref[...]
    xn_ref[...] = xn
    qkv = jax.lax.dot_general(xn.astype(jnp.bfloat16), w_ref[...],
                              (((1,), (1,)), ((), ())),
                              preferred_element_type=jnp.float32) + b_ref[...]
    q_ref[...] = qkv[:, :D].astype(jnp.bfloat16)
    k_ref[...] = qkv[:, D:2 * D].astype(jnp.bfloat16)
    v_ref[...] = qkv[:, 2 * D:].astype(jnp.bfloat16)


def _attn_body(q_ref, k_ref, v_ref, xn_ref, wo_ref, bo_ref, o_ref):
    q = q_ref[0]
    k = k_ref[0]
    s = jax.lax.dot_general(q, k, (((1,), (1,)), ((), ())),
                            preferred_element_type=jnp.float32) * _SCALE
    y = jax.lax.bitcast_convert_type(s, jnp.int32)
    y = jnp.where(y < 0, y ^ jnp.int32(0x7FFFFFFF), y)

    def _unmap(t):
        return jax.lax.bitcast_convert_type(
            jnp.where(t < 0, t ^ jnp.int32(0x7FFFFFFF), t), jnp.float32)

    lo = jnp.min(y, axis=-1, keepdims=True)
    hi = jnp.max(y, axis=-1, keepdims=True) + 1
    cl = jnp.full_like(lo, S)
    ch = jnp.zeros_like(lo)

    def cond(carry):
        _, lo, hi, cl, _ = carry
        done = (cl == K_KEEP) | ((hi - lo) == 1)
        return jnp.max((~done).astype(jnp.int32)) > 0

    def body(carry):
        it, lo, hi, cl, ch = carry
        lo_f = _unmap(lo)
        hi_f = _unmap(hi)
        frac = (cl - K_KEEP).astype(jnp.float32) / (cl - ch).astype(jnp.float32)
        mid_f = lo_f + (hi_f - lo_f) * frac
        mid_i = jax.lax.bitcast_convert_type(mid_f, jnp.int32)
        mid_interp = jnp.where(mid_i < 0, mid_i ^ jnp.int32(0x7FFFFFFF), mid_i)
        mid_bisect = (lo >> 1) + (hi >> 1) + ((lo | hi) & 1)
        mid = jnp.where((it & 3) != 3, mid_interp, mid_bisect)
        mid = jnp.minimum(jnp.maximum(mid, lo + 1), hi - 1)
        cnt = jnp.sum((y >= mid).astype(jnp.int32), axis=-1, keepdims=True)
        ge = cnt >= K_KEEP
        lo = jnp.where(ge, mid, lo)
        cl = jnp.where(ge, cnt, cl)
        hi = jnp.where(ge, hi, mid)
        ch = jnp.where(ge, ch, cnt)
        return it + 1, lo, hi, cl, ch

    _, lo, hi, cl, ch = jax.lax.while_loop(cond, body, (0, lo, hi, cl, ch))

    mask = y >= lo
    m = jnp.max(s, axis=-1, keepdims=True)
    p = jnp.where(mask, jnp.exp(s - m), 0.0)
    z = jnp.sum(p, axis=-1, keepdims=True)
    w = (p / z).astype(jnp.bfloat16)
    attn = jax.lax.dot_general(w, v_ref[0], (((1,), (0,)), ((), ())),
                               preferred_element_type=jnp.float32)
    out = jax.lax.dot_general(attn.astype(jnp.bfloat16), wo_ref[...],
                              (((1,), (1,)), ((), ())),
                              preferred_element_type=jnp.float32) + bo_ref[...]
    o_ref[0] = out + xn_ref[0]


@jax.jit
def kernel(x, Wqkv, bqkv, Wout, bout, gamma, beta):
    xf = x.reshape(B * S, D)
    xn, q, k, v = pl.pallas_call(
        _qkv_body,
        grid=(B * S // _RQ,),
        in_specs=[
            pl.BlockSpec((_RQ, D), lambda i: (i, 0)),
            pl.BlockSpec((3 * D, D), lambda i: (0, 0)),
            pl.BlockSpec((1, 3 * D), lambda i: (0, 0)),
            pl.BlockSpec((1, D), lambda i: (0, 0)),
            pl.BlockSpec((1, D), lambda i: (0, 0)),
        ],
        out_specs=[
            pl.BlockSpec((_RQ, D), lambda i: (i, 0)),
            pl.BlockSpec((_RQ, D), lambda i: (i, 0)),
            pl.BlockSpec((_RQ, D), lambda i: (i, 0)),
            pl.BlockSpec((_RQ, D), lambda i: (i, 0)),
        ],
        out_shape=[jax.ShapeDtypeStruct((B * S, D), jnp.float32)] +
                  [jax.ShapeDtypeStruct((B * S, D), jnp.bfloat16)] * 3,
    )(xf, Wqkv.astype(jnp.bfloat16), bqkv.reshape(1, 3 * D),
      gamma.reshape(1, D), beta.reshape(1, D))

    q = q.reshape(B, S, D)
    k = k.reshape(B, S, D)
    v = v.reshape(B, S, D)
    xn = xn.reshape(B, S, D)

    out = pl.pallas_call(
        _attn_body,
        grid=(B, S // _RA),
        in_specs=[
            pl.BlockSpec((1, _RA, D), lambda b, i: (b, i, 0)),
            pl.BlockSpec((1, S, D), lambda b, i: (b, 0, 0)),
            pl.BlockSpec((1, S, D), lambda b, i: (b, 0, 0)),
            pl.BlockSpec((1, _RA, D), lambda b, i: (b, i, 0)),
            pl.BlockSpec((D, D), lambda b, i: (0, 0)),
            pl.BlockSpec((1, D), lambda b, i: (0, 0)),
        ],
        out_specs=pl.BlockSpec((1, _RA, D), lambda b, i: (b, i, 0)),
        out_shape=jax.ShapeDtypeStruct((B, S, D), jnp.float32),
    )(q, k, v, xn, Wout.astype(jnp.bfloat16), bout.reshape(1, D))

    return out


# speedup vs baseline: 2.8517x; 1.3884x over previous
import functools

import jax
import jax.numpy as jnp
import numpy as np
from jax.experimental import pallas as pl
from jax.experimental.pallas import tpu as pltpu

D = 768
S = 2048
B = 4
K_KEEP = 614
_SCALE = 1.0 / np.sqrt(np.float32(D))

_RQ = 512
_RA = 256


def _qkv_body(x_ref, w_ref, b_ref, g_ref, be_ref, xn_ref, q_ref, k_ref, v_ref):
    x = x_ref[...]
    mu = jnp.mean(x, axis=-1, keepdims=True)
    var = jnp.mean((x - mu) * (x - mu), axis=-1, keepdims=True)
    xn = (x - mu) * jax.lax.rsqrt(var + 1e-5) * g_ref[...] + be_ref[...]
    xn_ref[...] = xn
    qkv = jax.lax.dot_general(xn.astype(jnp.bfloat16), w_ref[...],
                              (((1,), (1,)), ((), ())),
                              preferred_element_type=jnp.float32) + b_ref[...]
    q_ref[...] = qkv[:, :D].astype(jnp.bfloat16)
    k_ref[...] = qkv[:, D:2 * D].astype(jnp.bfloat16)
    v_ref[...] = qkv[:, 2 * D:].astype(jnp.bfloat16)


def _remap(f):
    i = jax.lax.bitcast_convert_type(f, jnp.int32)
    return jnp.where(i < 0, i ^ jnp.int32(0x7FFFFFFF), i)


def _unmap(t):
    return jax.lax.bitcast_convert_type(
        jnp.where(t < 0, t ^ jnp.int32(0x7FFFFFFF), t), jnp.float32)


_NFIX = 12


def _attn_body(q_ref, k_ref, v_ref, xn_ref, wo_ref, bo_ref, o_ref):
    q = q_ref[0]
    k = k_ref[0]
    s = jax.lax.dot_general(q, k, (((1,), (1,)), ((), ())),
                            preferred_element_type=jnp.float32) * _SCALE
    y = _remap(s)

    def count(t):
        return jnp.sum((y >= t).astype(jnp.int32), axis=-1, keepdims=True)

    m1 = jnp.mean(s, axis=-1, keepdims=True)
    m2 = jnp.mean(s * s, axis=-1, keepdims=True)
    sd = jnp.sqrt(jnp.maximum(m2 - m1 * m1, 0.0))
    ta = _remap(m1 + 0.44 * sd)
    tb = _remap(m1 + 0.61 * sd)
    ca = count(ta)
    cb = count(tb)
    lo0 = jnp.min(y, axis=-1, keepdims=True)
    hi0 = jnp.max(y, axis=-1, keepdims=True) + 1
    c_hi = cb >= K_KEEP
    c_mid = (~c_hi) & (ca >= K_KEEP)
    lo = jnp.where(c_hi, tb, jnp.where(c_mid, ta, lo0))
    cl = jnp.where(c_hi, cb, jnp.where(c_mid, ca, jnp.full_like(ca, S)))
    hi = jnp.where(c_hi, hi0, jnp.where(c_mid, tb, ta))
    ch = jnp.where(c_hi, jnp.zeros_like(ca), jnp.where(c_mid, cb, ca))

    def step(it, lo, hi, cl, ch):
        lo_f = _unmap(lo)
        hi_f = _unmap(hi)
        frac = (cl - K_KEEP).astype(jnp.float32) / (cl - ch).astype(jnp.float32)
        mid_interp = _remap(lo_f + (hi_f - lo_f) * frac)
        mid_bisect = (lo >> 1) + (hi >> 1) + ((lo | hi) & 1)
        mid = jnp.where((it & 3) != 3, mid_interp, mid_bisect)
        mid = jnp.minimum(jnp.maximum(mid, lo + 1), hi - 1)
        cnt = count(mid)
        ge = cnt >= K_KEEP
        lo = jnp.where(ge, mid, lo)
        cl = jnp.where(ge, cnt, cl)
        hi = jnp.where(ge, hi, mid)
        ch = jnp.where(ge, ch, cnt)
        return lo, hi, cl, ch

    def fbody(it, carry):
        return step(it, *carry)

    lo, hi, cl, ch = jax.lax.fori_loop(0, _NFIX, fbody, (lo, hi, cl, ch),
                                       unroll=True)

    def cond(carry):
        _, lo, hi, cl, _ = carry
        done = (cl == K_KEEP) | ((hi - lo) == 1)
        return jnp.max((~done).astype(jnp.int32)) > 0

    def wbody(carry):
        it, lo, hi, cl, ch = carry
        lo, hi, cl, ch = step(it, lo, hi, cl, ch)
        return it + 1, lo, hi, cl, ch

    _, lo, hi, cl, ch = jax.lax.while_loop(
        cond, wbody, (jnp.int32(_NFIX), lo, hi, cl, ch))

    mask = y >= lo
    m = jnp.max(s, axis=-1, keepdims=True)
    p = jnp.where(mask, jnp.exp(s - m), 0.0)
    z = jnp.sum(p, axis=-1, keepdims=True)
    w = (p / z).astype(jnp.bfloat16)
    attn = jax.lax.dot_general(w, v_ref[0], (((1,), (0,)), ((), ())),
                               preferred_element_type=jnp.float32)
    out = jax.lax.dot_general(attn.astype(jnp.bfloat16), wo_ref[...],
                              (((1,), (1,)), ((), ())),
                              preferred_element_type=jnp.float32) + bo_ref[...]
    o_ref[0] = out + xn_ref[0]


@jax.jit
def kernel(x, Wqkv, bqkv, Wout, bout, gamma, beta):
    xf = x.reshape(B * S, D)
    xn, q, k, v = pl.pallas_call(
        _qkv_body,
        grid=(B * S // _RQ,),
        in_specs=[
            pl.BlockSpec((_RQ, D), lambda i: (i, 0)),
            pl.BlockSpec((3 * D, D), lambda i: (0, 0)),
            pl.BlockSpec((1, 3 * D), lambda i: (0, 0)),
            pl.BlockSpec((1, D), lambda i: (0, 0)),
            pl.BlockSpec((1, D), lambda i: (0, 0)),
        ],
        out_specs=[
            pl.BlockSpec((_RQ, D), lambda i: (i, 0)),
            pl.BlockSpec((_RQ, D), lambda i: (i, 0)),
            pl.BlockSpec((_RQ, D), lambda i: (i, 0)),
            pl.BlockSpec((_RQ, D), lambda i: (i, 0)),
        ],
        out_shape=[jax.ShapeDtypeStruct((B * S, D), jnp.float32)] +
                  [jax.ShapeDtypeStruct((B * S, D), jnp.bfloat16)] * 3,
    )(xf, Wqkv.astype(jnp.bfloat16), bqkv.reshape(1, 3 * D),
      gamma.reshape(1, D), beta.reshape(1, D))

    q = q.reshape(B, S, D)
    k = k.reshape(B, S, D)
    v = v.reshape(B, S, D)
    xn = xn.reshape(B, S, D)

    out = pl.pallas_call(
        _attn_body,
        grid=(B, S // _RA),
        in_specs=[
            pl.BlockSpec((1, _RA, D), lambda b, i: (b, i, 0)),
            pl.BlockSpec((1, S, D), lambda b, i: (b, 0, 0)),
            pl.BlockSpec((1, S, D), lambda b, i: (b, 0, 0)),
            pl.BlockSpec((1, _RA, D), lambda b, i: (b, i, 0)),
            pl.BlockSpec((D, D), lambda b, i: (0, 0)),
            pl.BlockSpec((1, D), lambda b, i: (0, 0)),
        ],
        out_specs=pl.BlockSpec((1, _RA, D), lambda b, i: (b, i, 0)),
        out_shape=jax.ShapeDtypeStruct((B, S, D), jnp.float32),
    )(q, k, v, xn, Wout.astype(jnp.bfloat16), bout.reshape(1, D))

    return out


# speedup vs baseline: 2.8573x; 1.0019x over previous
import functools

import jax
import jax.numpy as jnp
import numpy as np
from jax.experimental import pallas as pl
from jax.experimental.pallas import tpu as pltpu

D = 768
S = 2048
B = 4
K_KEEP = 614
_SCALE = 1.0 / np.sqrt(np.float32(D))

_RQ = 512
_RA = 512


def _qkv_body(x_ref, w_ref, b_ref, g_ref, be_ref, xn_ref, q_ref, k_ref, v_ref):
    x = x_ref[...]
    mu = jnp.mean(x, axis=-1, keepdims=True)
    var = jnp.mean((x - mu) * (x - mu), axis=-1, keepdims=True)
    xn = (x - mu) * jax.lax.rsqrt(var + 1e-5) * g_ref[...] + be_ref[...]
    xn_ref[...] = xn
    qkv = jax.lax.dot_general(xn.astype(jnp.bfloat16), w_ref[...],
                              (((1,), (1,)), ((), ())),
                              preferred_element_type=jnp.float32) + b_ref[...]
    q_ref[...] = qkv[:, :D].astype(jnp.bfloat16)
    k_ref[...] = qkv[:, D:2 * D].astype(jnp.bfloat16)
    v_ref[...] = qkv[:, 2 * D:].astype(jnp.bfloat16)


def _remap(f):
    i = jax.lax.bitcast_convert_type(f, jnp.int32)
    return jnp.where(i < 0, i ^ jnp.int32(0x7FFFFFFF), i)


def _unmap(t):
    return jax.lax.bitcast_convert_type(
        jnp.where(t < 0, t ^ jnp.int32(0x7FFFFFFF), t), jnp.float32)


_NFIX = 12


def _attn_body(q_ref, k_ref, v_ref, xn_ref, wo_ref, bo_ref, o_ref):
    q = q_ref[0]
    k = k_ref[0]
    s = jax.lax.dot_general(q, k, (((1,), (1,)), ((), ())),
                            preferred_element_type=jnp.float32) * _SCALE
    y = _remap(s)

    def count(t):
        return jnp.sum((y >= t).astype(jnp.int32), axis=-1, keepdims=True)

    m1 = jnp.mean(s, axis=-1, keepdims=True)
    m2 = jnp.mean(s * s, axis=-1, keepdims=True)
    sd = jnp.sqrt(jnp.maximum(m2 - m1 * m1, 0.0))
    ta = _remap(m1 + 0.44 * sd)
    tb = _remap(m1 + 0.61 * sd)
    ca = count(ta)
    cb = count(tb)
    lo0 = jnp.min(y, axis=-1, keepdims=True)
    hi0 = jnp.max(y, axis=-1, keepdims=True) + 1
    c_hi = cb >= K_KEEP
    c_mid = (~c_hi) & (ca >= K_KEEP)
    lo = jnp.where(c_hi, tb, jnp.where(c_mid, ta, lo0))
    cl = jnp.where(c_hi, cb, jnp.where(c_mid, ca, jnp.full_like(ca, S)))
    hi = jnp.where(c_hi, hi0, jnp.where(c_mid, tb, ta))
    ch = jnp.where(c_hi, jnp.zeros_like(ca), jnp.where(c_mid, cb, ca))

    def step(it, lo, hi, cl, ch):
        lo_f = _unmap(lo)
        hi_f = _unmap(hi)
        frac = (cl - K_KEEP).astype(jnp.float32) / (cl - ch).astype(jnp.float32)
        mid_interp = _remap(lo_f + (hi_f - lo_f) * frac)
        mid_bisect = (lo >> 1) + (hi >> 1) + ((lo | hi) & 1)
        mid = jnp.where((it & 3) != 3, mid_interp, mid_bisect)
        mid = jnp.minimum(jnp.maximum(mid, lo + 1), hi - 1)
        cnt = count(mid)
        ge = cnt >= K_KEEP
        lo = jnp.where(ge, mid, lo)
        cl = jnp.where(ge, cnt, cl)
        hi = jnp.where(ge, hi, mid)
        ch = jnp.where(ge, ch, cnt)
        return lo, hi, cl, ch

    def fbody(it, carry):
        return step(it, *carry)

    lo, hi, cl, ch = jax.lax.fori_loop(0, _NFIX, fbody, (lo, hi, cl, ch),
                                       unroll=True)

    def cond(carry):
        _, lo, hi, cl, _ = carry
        done = (cl == K_KEEP) | ((hi - lo) == 1)
        return jnp.max((~done).astype(jnp.int32)) > 0

    def wbody(carry):
        it, lo, hi, cl, ch = carry
        lo, hi, cl, ch = step(it, lo, hi, cl, ch)
        return it + 1, lo, hi, cl, ch

    _, lo, hi, cl, ch = jax.lax.while_loop(
        cond, wbody, (jnp.int32(_NFIX), lo, hi, cl, ch))

    mask = y >= lo
    m = jnp.max(s, axis=-1, keepdims=True)
    p = jnp.where(mask, jnp.exp(s - m), 0.0)
    z = jnp.sum(p, axis=-1, keepdims=True)
    w = (p / z).astype(jnp.bfloat16)
    attn = jax.lax.dot_general(w, v_ref[0], (((1,), (0,)), ((), ())),
                               preferred_element_type=jnp.float32)
    out = jax.lax.dot_general(attn.astype(jnp.bfloat16), wo_ref[...],
                              (((1,), (1,)), ((), ())),
                              preferred_element_type=jnp.float32) + bo_ref[...]
    o_ref[0] = out + xn_ref[0]


@jax.jit
def kernel(x, Wqkv, bqkv, Wout, bout, gamma, beta):
    xf = x.reshape(B * S, D)
    xn, q, k, v = pl.pallas_call(
        _qkv_body,
        grid=(B * S // _RQ,),
        in_specs=[
            pl.BlockSpec((_RQ, D), lambda i: (i, 0)),
            pl.BlockSpec((3 * D, D), lambda i: (0, 0)),
            pl.BlockSpec((1, 3 * D), lambda i: (0, 0)),
            pl.BlockSpec((1, D), lambda i: (0, 0)),
            pl.BlockSpec((1, D), lambda i: (0, 0)),
        ],
        out_specs=[
            pl.BlockSpec((_RQ, D), lambda i: (i, 0)),
            pl.BlockSpec((_RQ, D), lambda i: (i, 0)),
            pl.BlockSpec((_RQ, D), lambda i: (i, 0)),
            pl.BlockSpec((_RQ, D), lambda i: (i, 0)),
        ],
        out_shape=[jax.ShapeDtypeStruct((B * S, D), jnp.float32)] +
                  [jax.ShapeDtypeStruct((B * S, D), jnp.bfloat16)] * 3,
    )(xf, Wqkv.astype(jnp.bfloat16), bqkv.reshape(1, 3 * D),
      gamma.reshape(1, D), beta.reshape(1, D))

    q = q.reshape(B, S, D)
    k = k.reshape(B, S, D)
    v = v.reshape(B, S, D)
    xn = xn.reshape(B, S, D)

    out = pl.pallas_call(
        _attn_body,
        grid=(B, S // _RA),
        in_specs=[
            pl.BlockSpec((1, _RA, D), lambda b, i: (b, i, 0)),
            pl.BlockSpec((1, S, D), lambda b, i: (b, 0, 0)),
            pl.BlockSpec((1, S, D), lambda b, i: (b, 0, 0)),
            pl.BlockSpec((1, _RA, D), lambda b, i: (b, i, 0)),
            pl.BlockSpec((D, D), lambda b, i: (0, 0)),
            pl.BlockSpec((1, D), lambda b, i: (0, 0)),
        ],
        out_specs=pl.BlockSpec((1, _RA, D), lambda b, i: (b, i, 0)),
        out_shape=jax.ShapeDtypeStruct((B, S, D), jnp.float32),
    )(q, k, v, xn, Wout.astype(jnp.bfloat16), bout.reshape(1, D))

    return out
